# baseline (speedup 1.0000x reference)
import functools

import jax
import jax.numpy as jnp
from jax import lax
from jax.experimental import pallas as pl
from jax.experimental.pallas import tpu as pltpu
from jax.experimental.pallas import tpu_sc as plsc

NC = 2
NS = 16
LN = 16
NW = NC * NS


def _mm_body(x_ref, w_ref, p_ref, wres_ref, xpa_ref, alr_ref, res_ref):
    xb = x_ref[...]
    xpb = jnp.dot(xb, w_ref[...], preferred_element_type=jnp.float32,
                  precision=lax.Precision.HIGHEST)
    alrb = jnp.dot(xpb, p_ref[...], preferred_element_type=jnp.float32,
                   precision=lax.Precision.HIGHEST)
    hc = xpb.shape[1]
    xpa_ref[:, :hc] = xpb
    xpa_ref[:, hc:] = alrb
    alr_ref[...] = alrb
    res_ref[...] = jnp.dot(xb, wres_ref[...], preferred_element_type=jnp.float32,
                           precision=lax.Precision.HIGHEST)


def _ep_body(a0_ref, a1_ref, res_ref, ex_ref, out_ref):
    hc = ex_ref.shape[1]
    h = ex_ref.shape[0]
    num = a0_ref[0, :, :hc] + a1_ref[0, :, :hc]
    den = a0_ref[0, :, hc:hc + h] + a1_ref[0, :, hc:hc + h]
    rec = 1.0 / (den + 1e-16)
    recf = jnp.dot(rec, ex_ref[...], preferred_element_type=jnp.float32)
    z = num * recf
    out_ref[...] = jnp.where(z > 0.0, z, jnp.exp(z) - 1.0) + res_ref[...]


def _vtake(row, idx):
    return lax.gather(
        row, idx[:, None],
        lax.GatherDimensionNumbers(offset_dims=(), collapsed_slice_dims=(0,),
                                   start_index_map=(0,)),
        slice_sizes=(1,), mode=lax.GatherScatterMode.PROMISE_IN_BOUNDS)


def kernel(x, edge_weight, W, att_l, att_r, W_res, edge_index):
    N, D = x.shape
    HC = W.shape[1]
    H = att_l.shape[1]
    C = att_l.shape[2]
    E = edge_index.shape[1]
    WD = HC + 2 * H
    f32 = jnp.float32

    eye = jnp.eye(H, dtype=f32)
    p_l = (att_l[0][:, :, None] * eye[:, None, :]).reshape(HC, H)
    p_r = (att_r[0][:, :, None] * eye[:, None, :]).reshape(HC, H)
    p_lr = jnp.concatenate([p_l, p_r], axis=1)
    ex = jnp.repeat(eye, C, axis=1)

    BN = 1000 if N % 1000 == 0 else 8
    grid_n = N // BN

    xpa, alr, res = pl.pallas_call(
        _mm_body,
        grid=(grid_n,),
        in_specs=[pl.BlockSpec((BN, D), lambda i: (i, 0)),
                  pl.BlockSpec((D, HC), lambda i: (0, 0)),
                  pl.BlockSpec((D, 2 * H), lambda i: (0, 0)),
                  pl.BlockSpec((D, HC), lambda i: (0, 0))],
        out_specs=[pl.BlockSpec((BN, WD), lambda i: (i, 0)),
                   pl.BlockSpec((BN, 2 * H), lambda i: (i, 0)),
                   pl.BlockSpec((BN, HC), lambda i: (i, 0))],
        out_shape=[jax.ShapeDtypeStruct((N, WD), f32),
                   jax.ShapeDtypeStruct((N, 2 * H), f32),
                   jax.ShapeDtypeStruct((N, HC), f32)],
    )(x, W, p_lr, W_res)

    K = 112
    EPT = -(-E // (NW * 2 * K)) * 2 * K
    CHUNKS = EPT // K
    EPAD = NW * EPT
    T0 = CHUNKS + 20
    T1 = 2 * CHUNKS - T0
    RPT = -(-(N + 1) // (NS * 8)) * 8
    NACC = NS * RPT

    pad = EPAD + 2 * K - E
    src = jnp.concatenate([edge_index[0], jnp.zeros((pad,), jnp.int32)])
    dst = jnp.concatenate([edge_index[1], jnp.full((pad,), N, jnp.int32)])
    ew = jnp.concatenate([edge_weight, jnp.zeros((pad,), f32)])
    pk = jnp.stack([src.reshape(-1, K), dst.reshape(-1, K)],
                   axis=1).reshape(NW * CHUNKS + 2, 2 * K)
    ew = ew.reshape(NW * CHUNKS + 2, K)
    zn = jnp.zeros((NACC, WD), f32)

    mesh = plsc.VectorSubcoreMesh(core_axis_name="c", subcore_axis_name="s")

    @functools.partial(
        pl.kernel,
        out_type=[jax.ShapeDtypeStruct((NC, NACC, WD), f32)],
        mesh=mesh,
        compiler_params=pltpu.CompilerParams(use_tc_tiling_on_sc=False),
        scratch_types=[
            pltpu.VMEM((2 * K,), jnp.int32),
            pltpu.VMEM((K,), f32),
            pltpu.VMEM((K, 2 * H), f32),
            pltpu.VMEM((K, WD), f32),
            pltpu.VMEM((K,), jnp.int32),
            pltpu.VMEM((2 * K,), jnp.int32),
            pltpu.VMEM((K,), f32),
            pltpu.VMEM((K, 2 * H), f32),
            pltpu.VMEM((K, WD), f32),
            pltpu.VMEM((K,), jnp.int32),
            pltpu.VMEM_SHARED((NACC, WD), f32),
            pltpu.SemaphoreType.DMA,
            pltpu.SemaphoreType.DMA,
            pltpu.SemaphoreType.DMA,
            pltpu.SemaphoreType.DMA,
            pltpu.SemaphoreType.DMA,
            pltpu.SemaphoreType.DMA,
        ],
    )
    def _sc_edge(xpa_hbm, alr_hbm, pk_hbm, ew_hbm, zn_hbm,
                 acc_out,
                 idxv0, ewv0, ald0, xpav0, dsts0,
                 idxv1, ewv1, ald1, xpav1, dsts1,
                 acc_sh, gs0, gs1, is0, is1, ss0, ss1):
        bufs = ((idxv0, ewv0, ald0, xpav0, dsts0, gs0, is0, ss0),
                (idxv1, ewv1, ald1, xpav1, dsts1, gs1, is1, ss1))
        cid = lax.axis_index("c")
        sid = lax.axis_index("s")
        nbase = sid * RPT
        pltpu.sync_copy(zn_hbm.at[pl.ds(nbase, RPT)],
                        acc_sh.at[pl.ds(nbase, RPT)])
        myT = jnp.where(cid == 0, T0, T1)
        cbase = jnp.where(cid == 0, sid * T0, NS * T0 + sid * T1)
        plsc.subcore_barrier()

        perm8 = jnp.arange(LN, dtype=jnp.int32) % H + H

        def issue_idx(b, i):
            idxv, ewv, isem = bufs[b][0], bufs[b][1], bufs[b][6]
            pltpu.async_copy(pk_hbm.at[cbase + i], idxv, isem)
            pltpu.async_copy(ew_hbm.at[cbase + i], ewv, isem)

        def wait_idx(b):
            idxv, ewv, isem = bufs[b][0], bufs[b][1], bufs[b][6]
            pltpu.make_async_copy(pk_hbm.at[cbase], idxv, isem).wait()
            pltpu.make_async_copy(ew_hbm.at[cbase], ewv, isem).wait()

        def issue_gathers(b):
            idxv, ald, xpav = bufs[b][0], bufs[b][2], bufs[b][3]
            gs = bufs[b][5]
            pltpu.async_copy(xpa_hbm.at[idxv.at[pl.ds(0, K)]], xpav, gs)
            pltpu.async_copy(alr_hbm.at[idxv.at[pl.ds(K, K)]], ald, gs)

        def wait_gathers(b):
            ald, xpav = bufs[b][2], bufs[b][3]
            gs = bufs[b][5]
            pltpu.make_async_copy(zn_hbm.at[pl.ds(0, K)], xpav, gs).wait()
            pltpu.make_async_copy(alr_hbm.at[pl.ds(0, K)], ald, gs).wait()

        def issue_scatter(b):
            xpav, dsts = bufs[b][3], bufs[b][4]
            ss = bufs[b][7]
            pltpu.async_copy(xpav, acc_sh.at[dsts], ss, add=True)

        def wait_scatter(b):
            xpav, ss = bufs[b][3], bufs[b][7]
            pltpu.make_async_copy(zn_hbm.at[pl.ds(0, K)], xpav, ss).wait()

        def compute(b):
            idxv, ewv, ald, xpav, dsts = (bufs[b][0], bufs[b][1], bufs[b][2],
                                          bufs[b][3], bufs[b][4])

            def group(g, cc):
                ewb = ewv[pl.ds(g * LN, LN)]
                dsts[pl.ds(g * LN, LN)] = idxv[pl.ds(K + g * LN, LN)]
                for j in range(LN):
                    e = g * LN + j
                    va = xpav[e, pl.ds(HC, LN)]
                    vb = ald[e, :]
                    asum = va + _vtake(vb, perm8)
                    ews = _vtake(ewb, jnp.full((LN,), j, jnp.int32))
                    a = ews * asum
                    a = jnp.where(a >= 0.0, a, 0.2 * a)
                    srow = jnp.exp(a)
                    xpav[e, pl.ds(HC, LN)] = srow
                    for h in range(H):
                        sh = _vtake(srow, jnp.full((LN,), h, jnp.int32))
                        xpav[e, pl.ds(h * C, C)] = xpav[e, pl.ds(h * C, C)] * sh
                return cc

            lax.fori_loop(0, K // LN, group, 0)

        issue_idx(0, 0)
        wait_idx(0)
        issue_gathers(0)
        issue_idx(1, 1)
        pltpu.async_copy(zn_hbm.at[pl.ds(0, K)], bufs[1][3], bufs[1][7])

        def pair(t, carry):
            for b in range(2):
                i = t * 2 + b
                o = 1 - b
                wait_gathers(b)
                wait_scatter(o)
                wait_idx(o)
                issue_gathers(o)
                compute(b)
                issue_scatter(b)
                issue_idx(b, i + 2)
            return carry

        lax.fori_loop(0, myT // 2, pair, 0)
        wait_scatter(1)
        wait_gathers(0)
        wait_idx(1)
        plsc.subcore_barrier()
        pltpu.sync_copy(acc_sh.at[pl.ds(nbase, RPT)],
                        acc_out.at[cid, pl.ds(nbase, RPT)])

    (acc2,) = _sc_edge(xpa, alr, pk, ew, zn)

    out = pl.pallas_call(
        _ep_body,
        grid=(grid_n,),
        in_specs=[pl.BlockSpec((1, BN, WD), lambda i: (0, i, 0)),
                  pl.BlockSpec((1, BN, WD), lambda i: (1, i, 0)),
                  pl.BlockSpec((BN, HC), lambda i: (i, 0)),
                  pl.BlockSpec((H, HC), lambda i: (0, 0))],
        out_specs=pl.BlockSpec((BN, HC), lambda i: (i, 0)),
        out_shape=jax.ShapeDtypeStruct((N, HC), f32),
    )(acc2, acc2, res, ex)
    return out


# speedup vs baseline: 1.0390x; 1.0390x over previous
import functools

import jax
import jax.numpy as jnp
from jax import lax
from jax.experimental import pallas as pl
from jax.experimental.pallas import tpu as pltpu
from jax.experimental.pallas import tpu_sc as plsc

NC = 2
NS = 16
LN = 16
NW = NC * NS


def _mm_body(x_ref, w_ref, p_ref, wres_ref, xpa_ref, alr_ref, res_ref):
    xb = x_ref[...]
    xpb = jnp.dot(xb, w_ref[...], preferred_element_type=jnp.float32,
                  precision=lax.Precision.HIGHEST)
    alrb = jnp.dot(xpb, p_ref[...], preferred_element_type=jnp.float32,
                   precision=lax.Precision.HIGHEST)
    hc = xpb.shape[1]
    xpa_ref[:, :hc] = xpb
    xpa_ref[:, hc:] = alrb
    alr_ref[...] = alrb
    res_ref[...] = jnp.dot(xb, wres_ref[...], preferred_element_type=jnp.float32,
                           precision=lax.Precision.HIGHEST)


def _ep_body(a0_ref, a1_ref, res_ref, ex_ref, out_ref):
    hc = ex_ref.shape[1]
    h = ex_ref.shape[0]
    num = a0_ref[0, :, :hc] + a1_ref[0, :, :hc]
    den = a0_ref[0, :, hc:hc + h] + a1_ref[0, :, hc:hc + h]
    rec = 1.0 / (den + 1e-16)
    recf = jnp.dot(rec, ex_ref[...], preferred_element_type=jnp.float32)
    z = num * recf
    out_ref[...] = jnp.where(z > 0.0, z, jnp.exp(z) - 1.0) + res_ref[...]


def _vtake(row, idx):
    return lax.gather(
        row, idx[:, None],
        lax.GatherDimensionNumbers(offset_dims=(), collapsed_slice_dims=(0,),
                                   start_index_map=(0,)),
        slice_sizes=(1,), mode=lax.GatherScatterMode.PROMISE_IN_BOUNDS)


def kernel(x, edge_weight, W, att_l, att_r, W_res, edge_index):
    N, D = x.shape
    HC = W.shape[1]
    H = att_l.shape[1]
    C = att_l.shape[2]
    E = edge_index.shape[1]
    WD = HC + 2 * H
    f32 = jnp.float32

    eye = jnp.eye(H, dtype=f32)
    p_l = (att_l[0][:, :, None] * eye[:, None, :]).reshape(HC, H)
    p_r = (att_r[0][:, :, None] * eye[:, None, :]).reshape(HC, H)
    p_lr = jnp.concatenate([p_l, p_r], axis=1)
    ex = jnp.repeat(eye, C, axis=1)

    BN = 1000 if N % 1000 == 0 else 8
    grid_n = N // BN

    xpa, alr, res = pl.pallas_call(
        _mm_body,
        grid=(grid_n,),
        in_specs=[pl.BlockSpec((BN, D), lambda i: (i, 0)),
                  pl.BlockSpec((D, HC), lambda i: (0, 0)),
                  pl.BlockSpec((D, 2 * H), lambda i: (0, 0)),
                  pl.BlockSpec((D, HC), lambda i: (0, 0))],
        out_specs=[pl.BlockSpec((BN, WD), lambda i: (i, 0)),
                   pl.BlockSpec((BN, 2 * H), lambda i: (i, 0)),
                   pl.BlockSpec((BN, HC), lambda i: (i, 0))],
        out_shape=[jax.ShapeDtypeStruct((N, WD), f32),
                   jax.ShapeDtypeStruct((N, 2 * H), f32),
                   jax.ShapeDtypeStruct((N, HC), f32)],
    )(x, W, p_lr, W_res)

    K = 112
    EPT = -(-E // (NW * 2 * K)) * 2 * K
    CHUNKS = EPT // K
    EPAD = NW * EPT
    T0 = CHUNKS + 14
    T1 = 2 * CHUNKS - T0
    RPT = -(-(N + 1) // (NS * 8)) * 8
    NACC = NS * RPT

    pad = EPAD + 2 * K - E
    src = jnp.concatenate([edge_index[0], jnp.zeros((pad,), jnp.int32)])
    dst = jnp.concatenate([edge_index[1], jnp.full((pad,), N, jnp.int32)])
    ew = jnp.concatenate([edge_weight, jnp.zeros((pad,), f32)])
    pk = jnp.stack([src.reshape(-1, K), dst.reshape(-1, K)],
                   axis=1).reshape(NW * CHUNKS + 2, 2 * K)
    ew = ew.reshape(NW * CHUNKS + 2, K)
    zn = jnp.zeros((NACC, WD), f32)

    mesh = plsc.VectorSubcoreMesh(core_axis_name="c", subcore_axis_name="s")

    @functools.partial(
        pl.kernel,
        out_type=[jax.ShapeDtypeStruct((NC, NACC, WD), f32)],
        mesh=mesh,
        compiler_params=pltpu.CompilerParams(use_tc_tiling_on_sc=False),
        scratch_types=[
            pltpu.VMEM((2 * K,), jnp.int32),
            pltpu.VMEM((K,), f32),
            pltpu.VMEM((K, 2 * H), f32),
            pltpu.VMEM((K, WD), f32),
            pltpu.VMEM((K,), jnp.int32),
            pltpu.VMEM((2 * K,), jnp.int32),
            pltpu.VMEM((K,), f32),
            pltpu.VMEM((K, 2 * H), f32),
            pltpu.VMEM((K, WD), f32),
            pltpu.VMEM((K,), jnp.int32),
            pltpu.VMEM_SHARED((NACC, WD), f32),
            pltpu.SemaphoreType.DMA,
            pltpu.SemaphoreType.DMA,
            pltpu.SemaphoreType.DMA,
            pltpu.SemaphoreType.DMA,
            pltpu.SemaphoreType.DMA,
            pltpu.SemaphoreType.DMA,
        ],
    )
    def _sc_edge(xpa_hbm, alr_hbm, pk_hbm, ew_hbm, zn_hbm,
                 acc_out,
                 idxv0, ewv0, ald0, xpav0, dsts0,
                 idxv1, ewv1, ald1, xpav1, dsts1,
                 acc_sh, gs0, gs1, is0, is1, ss0, ss1):
        bufs = ((idxv0, ewv0, ald0, xpav0, dsts0, gs0, is0, ss0),
                (idxv1, ewv1, ald1, xpav1, dsts1, gs1, is1, ss1))
        cid = lax.axis_index("c")
        sid = lax.axis_index("s")
        nbase = sid * RPT
        pltpu.sync_copy(zn_hbm.at[pl.ds(nbase, RPT)],
                        acc_sh.at[pl.ds(nbase, RPT)])
        myT = jnp.where(cid == 0, T0, T1)
        cbase = jnp.where(cid == 0, sid * T0, NS * T0 + sid * T1)
        plsc.subcore_barrier()

        perm8 = jnp.arange(LN, dtype=jnp.int32) % H + H

        def issue_idx(b, i):
            idxv, ewv, isem = bufs[b][0], bufs[b][1], bufs[b][6]
            pltpu.async_copy(pk_hbm.at[cbase + i], idxv, isem)
            pltpu.async_copy(ew_hbm.at[cbase + i], ewv, isem)

        def wait_idx(b):
            idxv, ewv, isem = bufs[b][0], bufs[b][1], bufs[b][6]
            pltpu.make_async_copy(pk_hbm.at[cbase], idxv, isem).wait()
            pltpu.make_async_copy(ew_hbm.at[cbase], ewv, isem).wait()

        def issue_gathers(b):
            idxv, ald, xpav = bufs[b][0], bufs[b][2], bufs[b][3]
            gs = bufs[b][5]
            pltpu.async_copy(xpa_hbm.at[idxv.at[pl.ds(0, K)]], xpav, gs)
            pltpu.async_copy(alr_hbm.at[idxv.at[pl.ds(K, K)]], ald, gs)

        def wait_gathers(b):
            ald, xpav = bufs[b][2], bufs[b][3]
            gs = bufs[b][5]
            pltpu.make_async_copy(zn_hbm.at[pl.ds(0, K)], xpav, gs).wait()
            pltpu.make_async_copy(alr_hbm.at[pl.ds(0, K)], ald, gs).wait()

        def issue_scatter(b):
            xpav, dsts = bufs[b][3], bufs[b][4]
            ss = bufs[b][7]
            pltpu.async_copy(xpav, acc_sh.at[dsts], ss, add=True)

        def wait_scatter(b):
            xpav, ss = bufs[b][3], bufs[b][7]
            pltpu.make_async_copy(zn_hbm.at[pl.ds(0, K)], xpav, ss).wait()

        def compute(b):
            idxv, ewv, ald, xpav, dsts = (bufs[b][0], bufs[b][1], bufs[b][2],
                                          bufs[b][3], bufs[b][4])

            def group(g, cc):
                ewb = ewv[pl.ds(g * LN, LN)]
                dsts[pl.ds(g * LN, LN)] = idxv[pl.ds(K + g * LN, LN)]
                for j in range(LN):
                    e = g * LN + j
                    va = xpav[e, pl.ds(HC, LN)]
                    vb = ald[e, :]
                    asum = va + _vtake(vb, perm8)
                    ews = _vtake(ewb, jnp.full((LN,), j, jnp.int32))
                    a = ews * asum
                    a = jnp.where(a >= 0.0, a, 0.2 * a)
                    srow = jnp.exp(a)
                    xpav[e, pl.ds(HC, LN)] = srow
                    for h in range(H):
                        sh = _vtake(srow, jnp.full((LN,), h, jnp.int32))
                        xpav[e, pl.ds(h * C, C)] = xpav[e, pl.ds(h * C, C)] * sh
                return cc

            lax.fori_loop(0, K // LN, group, 0)

        issue_idx(0, 0)
        wait_idx(0)
        issue_gathers(0)
        issue_idx(1, 1)
        pltpu.async_copy(zn_hbm.at[pl.ds(0, K)], bufs[1][3], bufs[1][7])

        def pair(t, carry):
            for b in range(2):
                i = t * 2 + b
                o = 1 - b
                wait_gathers(b)
                wait_scatter(o)
                wait_idx(o)
                issue_gathers(o)
                compute(b)
                issue_scatter(b)
                issue_idx(b, i + 2)
            return carry

        lax.fori_loop(0, myT // 2, pair, 0)
        wait_scatter(1)
        wait_gathers(0)
        wait_idx(1)
        plsc.subcore_barrier()
        pltpu.sync_copy(acc_sh.at[pl.ds(nbase, RPT)],
                        acc_out.at[cid, pl.ds(nbase, RPT)])

    (acc2,) = _sc_edge(xpa, alr, pk, ew, zn)

    out = pl.pallas_call(
        _ep_body,
        grid=(grid_n,),
        in_specs=[pl.BlockSpec((1, BN, WD), lambda i: (0, i, 0)),
                  pl.BlockSpec((1, BN, WD), lambda i: (1, i, 0)),
                  pl.BlockSpec((BN, HC), lambda i: (i, 0)),
                  pl.BlockSpec((H, HC), lambda i: (0, 0))],
        out_specs=pl.BlockSpec((BN, HC), lambda i: (i, 0)),
        out_shape=jax.ShapeDtypeStruct((N, HC), f32),
    )(acc2, acc2, res, ex)
    return out
